# baseline (speedup 1.0000x reference)
import jax
import jax.numpy as jnp
from jax.experimental import pallas as pl
from jax.experimental.pallas import tpu as pltpu

B, C, T = 128, 16, 16384


def _body(w_ref, s_ref, x_ref, o_ref, sem):
    pltpu.async_copy(x_ref, o_ref, sem).wait()


def kernel(x, widths, starts):
    return pl.pallas_call(
        _body,
        out_shape=jax.ShapeDtypeStruct((B, C, T), jnp.float32),
        in_specs=[
            pl.BlockSpec(memory_space=pltpu.SMEM),
            pl.BlockSpec(memory_space=pltpu.SMEM),
            pl.BlockSpec(memory_space=pl.ANY),
        ],
        out_specs=pl.BlockSpec(memory_space=pl.ANY),
        scratch_shapes=[pltpu.SemaphoreType.DMA],
    )(widths, starts, x)


# speedup vs baseline: 48.9937x; 48.9937x over previous
import jax
import jax.numpy as jnp
from jax import lax
from jax.experimental import pallas as pl
from jax.experimental.pallas import tpu as pltpu

B, C, T = 128, 16, 16384
M = 2
BS = 8
WINW = 384


def _tc_body(w_ref, s_ref, x_ref, o_ref):
    g = pl.program_id(0)
    o_ref[...] = x_ref[...]
    pos = lax.broadcasted_iota(jnp.int32, (C, WINW), 1)
    for j in range(BS):
        b = g * BS + j
        for m in range(M):
            s = s_ref[m, b]
            e = jnp.minimum(s + w_ref[m, b], T)
            win = pl.multiple_of(
                jnp.minimum((s // 128) * 128, T - WINW), 128)
            p = pos + win
            keep = (p < s) | (p >= e)
            chunk = o_ref[j, :, pl.ds(win, WINW)]
            o_ref[j, :, pl.ds(win, WINW)] = jnp.where(keep, chunk, 0.0)


def kernel(x, widths, starts):
    return pl.pallas_call(
        _tc_body,
        out_shape=jax.ShapeDtypeStruct((B, C, T), jnp.float32),
        grid=(B // BS,),
        in_specs=[
            pl.BlockSpec(memory_space=pltpu.SMEM),
            pl.BlockSpec(memory_space=pltpu.SMEM),
            pl.BlockSpec((BS, C, T), lambda g: (g, 0, 0)),
        ],
        out_specs=pl.BlockSpec((BS, C, T), lambda g: (g, 0, 0)),
    )(widths, starts, x)
